# baseline (speedup 1.0000x reference)
import functools

import jax
import jax.numpy as jnp
from jax import lax
from jax.experimental import pallas as pl
from jax.experimental.pallas import tpu as pltpu
from jax.experimental.pallas import tpu_sc as plsc

_N_NODES = 1000000
_EMB = 64
_HALF = _EMB // 2
_IN_DIM = 128
_B = 16384
_OUT_DIM = _IN_DIM + _EMB
_NROWS = _N_NODES + 1

_NC = 2
_NS = 16
_NW = _NC * _NS
_BPW = _B // _NW


_CB = 8192


def _rne_bf16_hi(u):
    return (u + jnp.uint32(0x7FFF) + ((u >> 16) & jnp.uint32(1))) & jnp.uint32(
        0xFFFF0000
    )


def _proj_body(tt_ref, w_ref, out_ref):
    e = lax.dot_general(
        tt_ref[...], w_ref[...],
        dimension_numbers=(((0,), (1,)), ((), ())),
        preferred_element_type=jnp.float32,
    )
    u = lax.bitcast_convert_type(e, jnp.uint32)
    lo = _rne_bf16_hi(u[:, :_HALF]) >> 16
    hi = _rne_bf16_hi(u[:, _HALF:])
    out_ref[...] = lax.bitcast_convert_type(lo | hi, jnp.float32)


def _project_table(table_t, W):
    steps = (_NROWS + _CB - 1) // _CB
    return pl.pallas_call(
        _proj_body,
        grid=(steps,),
        in_specs=[
            pl.BlockSpec((_EMB, _CB), lambda i: (0, i)),
            pl.BlockSpec((_EMB, _EMB), lambda i: (0, 0)),
        ],
        out_specs=pl.BlockSpec((_CB, _HALF), lambda i: (i, 0)),
        out_shape=jax.ShapeDtypeStruct((_NROWS, _HALF), jnp.float32),
    )(table_t, W)


@functools.partial(
    pl.kernel,
    mesh=plsc.VectorSubcoreMesh(core_axis_name="c", subcore_axis_name="s"),
    out_type=jax.ShapeDtypeStruct((_B, _HALF), jnp.float32),
    scratch_types=[
        pltpu.VMEM((_BPW,), jnp.int32),
        pltpu.VMEM((_BPW, _HALF), jnp.float32),
        pltpu.SemaphoreType.DMA,
    ],
)
def _sc_gather(idx_hbm, proj_hbm, out_hbm, idx_v, rows_v, sem):
    wid = lax.axis_index("s") * _NC + lax.axis_index("c")
    base = wid * _BPW
    pltpu.sync_copy(idx_hbm.at[pl.ds(base, _BPW)], idx_v)

    def body(g, carry):
        vec = idx_v[pl.ds(g * 16, 16)]
        for j in range(16):
            r = vec[j]
            pltpu.async_copy(
                proj_hbm.at[pl.ds(r, 1)],
                rows_v.at[pl.ds(g * 16 + j, 1)],
                sem,
            )
        return carry

    lax.fori_loop(0, _BPW // 16, body, 0)
    pltpu.make_async_copy(proj_hbm.at[pl.ds(0, _BPW)], rows_v, sem).wait()
    pltpu.sync_copy(rows_v, out_hbm.at[pl.ds(base, _BPW)])


_RB = 2048


def _tc_body(feats_ref, g_ref, b_ref, out_ref):
    u = lax.bitcast_convert_type(g_ref[...], jnp.uint32)
    lo = lax.bitcast_convert_type(u << 16, jnp.float32)
    hi = lax.bitcast_convert_type(u & jnp.uint32(0xFFFF0000), jnp.float32)
    out_ref[:, :_IN_DIM] = feats_ref[...]
    out_ref[:, _IN_DIM:_IN_DIM + _HALF] = lo + b_ref[:, :_HALF]
    out_ref[:, _IN_DIM + _HALF:] = hi + b_ref[:, _HALF:]


def _tc_fused(feats, gathered, b2):
    return pl.pallas_call(
        _tc_body,
        grid=(_B // _RB,),
        in_specs=[
            pl.BlockSpec((_RB, _IN_DIM), lambda i: (i, 0)),
            pl.BlockSpec((_RB, _HALF), lambda i: (i, 0)),
            pl.BlockSpec((1, _EMB), lambda i: (0, 0)),
        ],
        out_specs=pl.BlockSpec((_RB, _OUT_DIM), lambda i: (i, 0)),
        out_shape=jax.ShapeDtypeStruct((_B, _OUT_DIM), jnp.float32),
    )(feats, gathered, b2)


def kernel(ids, feats, layer_idx, table, W, b):
    lookup = jnp.where(layer_idx > 0, ids, _N_NODES).astype(jnp.int32)
    projected = _project_table(table.T, W)
    gathered = _sc_gather(lookup, projected)
    return _tc_fused(feats, gathered, b.reshape(1, _EMB))


# speedup vs baseline: 1.1431x; 1.1431x over previous
import functools

import jax
import jax.numpy as jnp
from jax import lax
from jax.experimental import pallas as pl
from jax.experimental.pallas import tpu as pltpu
from jax.experimental.pallas import tpu_sc as plsc

_N_NODES = 1000000
_EMB = 64
_IN_DIM = 128
_B = 16384
_OUT_DIM = _IN_DIM + _EMB
_NROWS = _N_NODES + 1

_NC = 2
_NS = 16
_NW = _NC * _NS
_BPW = _B // _NW


_CB = 8192


def _proj_body(tt_ref, w_ref, out_ref):
    out_ref[...] = lax.dot_general(
        tt_ref[...], w_ref[...],
        dimension_numbers=(((0,), (1,)), ((), ())),
        preferred_element_type=jnp.float32,
    )


def _project_table(table_t, W):
    steps = (_NROWS + _CB - 1) // _CB
    return pl.pallas_call(
        _proj_body,
        grid=(steps,),
        in_specs=[
            pl.BlockSpec((_EMB, _CB), lambda i: (0, i)),
            pl.BlockSpec((_EMB, _EMB), lambda i: (0, 0)),
        ],
        out_specs=pl.BlockSpec((_CB, _EMB), lambda i: (i, 0)),
        out_shape=jax.ShapeDtypeStruct((_NROWS, _EMB), jnp.float32),
    )(table_t, W)


@functools.partial(
    pl.kernel,
    mesh=plsc.VectorSubcoreMesh(core_axis_name="c", subcore_axis_name="s"),
    out_type=jax.ShapeDtypeStruct((_B, _EMB), jnp.float32),
    scratch_types=[
        pltpu.VMEM((_BPW,), jnp.int32),
        pltpu.VMEM((_BPW, _EMB), jnp.float32),
        pltpu.SemaphoreType.DMA,
    ],
)
def _sc_gather(idx_hbm, proj_hbm, out_hbm, idx_v, rows_v, sem):
    wid = lax.axis_index("s") * _NC + lax.axis_index("c")
    base = wid * _BPW
    pltpu.sync_copy(idx_hbm.at[pl.ds(base, _BPW)], idx_v)

    def body(g, carry):
        vec = idx_v[pl.ds(g * 16, 16)]
        for j in range(16):
            r = vec[j]
            pltpu.async_copy(
                proj_hbm.at[pl.ds(r, 1)],
                rows_v.at[pl.ds(g * 16 + j, 1)],
                sem,
            )
        return carry

    lax.fori_loop(0, _BPW // 16, body, 0)
    pltpu.make_async_copy(proj_hbm.at[pl.ds(0, _BPW)], rows_v, sem).wait()
    pltpu.sync_copy(rows_v, out_hbm.at[pl.ds(base, _BPW)])


_RB = 2048


def _tc_body(feats_ref, g_ref, i128_ref, i64_ref, bt_ref, out_ref):
    ft = lax.dot_general(
        i128_ref[...], feats_ref[...],
        dimension_numbers=(((1,), (1,)), ((), ())),
        preferred_element_type=jnp.float32,
    )
    gt = lax.dot_general(
        i64_ref[...], g_ref[...],
        dimension_numbers=(((1,), (1,)), ((), ())),
        preferred_element_type=jnp.float32,
    )
    out_ref[:_IN_DIM, :] = ft
    out_ref[_IN_DIM:, :] = gt + bt_ref[...]


def _tc_fused(feats, gathered, bt):
    i128 = jnp.eye(_IN_DIM, dtype=jnp.float32)
    i64 = jnp.eye(_EMB, dtype=jnp.float32)
    out_t = pl.pallas_call(
        _tc_body,
        grid=(_B // _RB,),
        in_specs=[
            pl.BlockSpec((_RB, _IN_DIM), lambda i: (i, 0)),
            pl.BlockSpec((_RB, _EMB), lambda i: (i, 0)),
            pl.BlockSpec((_IN_DIM, _IN_DIM), lambda i: (0, 0)),
            pl.BlockSpec((_EMB, _EMB), lambda i: (0, 0)),
            pl.BlockSpec((_EMB, 1), lambda i: (0, 0)),
        ],
        out_specs=pl.BlockSpec((_OUT_DIM, _RB), lambda i: (0, i)),
        out_shape=jax.ShapeDtypeStruct((_OUT_DIM, _B), jnp.float32),
    )(feats, gathered, i128, i64, bt)
    return out_t.T


def kernel(ids, feats, layer_idx, table, W, b):
    lookup = jnp.where(layer_idx > 0, ids, _N_NODES).astype(jnp.int32)
    projected = _project_table(table.T, W)
    gathered = _sc_gather(lookup, projected)
    return _tc_fused(feats, gathered, b.reshape(_EMB, 1))
